# Initial kernel scaffold; baseline (speedup 1.0000x reference)
import jax
import jax.numpy as jnp
from jax.experimental import pallas as pl


def kernel(non_seq_ids, seq_ids, non_seq_table, seq_table):
    raise NotImplementedError("write your pallas kernel here")


# speedup vs baseline: 3.4858x; 3.4858x over previous
import functools

import jax
import jax.numpy as jnp
from jax import lax
from jax.experimental import pallas as pl
from jax.experimental.pallas import tpu as pltpu
from jax.experimental.pallas import tpu_sc as plsc

B = 4096
F_NS = 26
F_S = 4
V = 100000
D = 4
L = 50

NC = 2
NSUB = 16
NW = NC * NSUB
BPW = B // NW
NSI = BPW * F_NS
SQS = BPW * F_S
SQI = SQS * L
D_NS = F_NS * D
D_S = F_S * D

CH = 5
NCHUNK = L // CH
NGRP = SQS // 4


def _build():
  mesh = plsc.VectorSubcoreMesh(core_axis_name="c", subcore_axis_name="s")

  @functools.partial(
      pl.kernel,
      out_type=(
          jax.ShapeDtypeStruct((B * F_NS, D), jnp.float32),
          jax.ShapeDtypeStruct((B * F_S, D), jnp.float32),
      ),
      mesh=mesh,
      compiler_params=pltpu.CompilerParams(
          needs_layout_passes=False, use_tc_tiling_on_sc=False
      ),
      scratch_types=[
          pltpu.VMEM((NSI,), jnp.int32),
          pltpu.VMEM((SQI,), jnp.int32),
          pltpu.VMEM((NSI, D), jnp.float32),
          pltpu.VMEM((CH * SQS, D), jnp.float32),
          pltpu.VMEM((CH * SQS, D), jnp.float32),
          pltpu.VMEM((SQS, D), jnp.float32),
          pltpu.SemaphoreType.DMA,
          pltpu.SemaphoreType.DMA,
          pltpu.SemaphoreType.DMA,
      ],
  )
  def run(tbl_ns, tbl_s, idx_ns, idx_s, out_ns, out_s, insv, isv, rows_ns,
          rows_a, rows_b, acc, sem_ns, sem_a, sem_b):
    wid = lax.axis_index("s") * NC + lax.axis_index("c")

    pltpu.sync_copy(idx_ns.at[pl.ds(wid * NSI, NSI)], insv)
    cp_ns = pltpu.async_copy(tbl_ns.at[insv], rows_ns, sem_ns)
    pltpu.sync_copy(idx_s.at[pl.ds(wid * SQI, SQI)], isv)

    bufs = (rows_a, rows_b)
    sems = (sem_a, sem_b)

    lane = lax.iota(jnp.int32, 16)
    rig = lax.shift_right_logical(lane, 2)
    col = lax.bitwise_and(lane, 3)
    scale = jnp.full((16,), 1.0 / L, dtype=jnp.float32)

    cp = pltpu.async_copy(
        tbl_s.at[isv.at[pl.ds(0, CH * SQS)]], bufs[0], sems[0]
    )
    for c in range(NCHUNK):
      cp.wait()
      if c + 1 < NCHUNK:
        cp = pltpu.async_copy(
            tbl_s.at[isv.at[pl.ds((c + 1) * CH * SQS, CH * SQS)]],
            bufs[(c + 1) % 2],
            sems[(c + 1) % 2],
        )
      buf = bufs[c % 2]
      first = c == 0
      last = c == NCHUNK - 1

      def grp_body(j, _, buf=buf, first=first, last=last):
        rows0 = rig + 4 * j
        if first:
          a = plsc.load_gather(buf, [rows0, col])
          start = 1
        else:
          a = plsc.load_gather(acc, [rows0, col])
          start = 0
        for step in range(start, CH):
          a = a + plsc.load_gather(buf, [rows0 + step * SQS, col])
        if last:
          a = a * scale
        plsc.store_scatter(acc, [rows0, col], a)
        return _

      lax.fori_loop(0, NGRP, grp_body, None)

    cp_ns.wait()
    pltpu.sync_copy(rows_ns, out_ns.at[pl.ds(wid * NSI, NSI)])
    pltpu.sync_copy(acc, out_s.at[pl.ds(wid * SQS, SQS)])

  return run


_run = _build()


@jax.jit
def kernel(non_seq_ids, seq_ids, non_seq_table, seq_table):
  flat_ns = non_seq_table.reshape(F_NS * V, D)
  flat_s = seq_table.reshape(F_S * V, D)
  idx_ns = (
      non_seq_ids + (jnp.arange(F_NS, dtype=jnp.int32) * V)[None, :]
  ).reshape(B * F_NS)
  idx_s = seq_ids + (jnp.arange(F_S, dtype=jnp.int32) * V)[None, :, None]
  idx_s = (
      idx_s.reshape(NW, BPW, F_S, L).transpose(0, 3, 1, 2).reshape(NW * SQI)
  )
  out_ns, out_s = _run(flat_ns, flat_s, idx_ns, idx_s)
  return jnp.concatenate(
      [out_ns.reshape(B, D_NS), out_s.reshape(B, D_S)], axis=1
  )
